# baseline (speedup 1.0000x reference)
import functools

import jax
import jax.numpy as jnp
from jax import lax
from jax.experimental import pallas as pl
from jax.experimental.pallas import tpu as pltpu
from jax.experimental.pallas import tpu_sc as plsc

N = 10000
K = 32
D = 128

NC = 2
NS = 16
NW = NC * NS
L = 16

R = 320
NP = NW * R
E = N * K
CH = 8192
NCH = -(-E // CH)
EP = NCH * CH
G = 256
CHA = 64


def _matmul_body(x_ref, wc_ref, bc_ref, out_ref):
    out_ref[...] = (
        jnp.dot(x_ref[...], wc_ref[...], preferred_element_type=jnp.float32)
        + bc_ref[...]
    )


def _projections(x, Wc, bc):
    grid = 10
    blk = N // grid
    return pl.pallas_call(
        _matmul_body,
        grid=(grid,),
        in_specs=[
            pl.BlockSpec((blk, D), lambda i: (i, 0)),
            pl.BlockSpec((D, 2 * D), lambda i: (0, 0)),
            pl.BlockSpec((1, 2 * D), lambda i: (0, 0)),
        ],
        out_specs=pl.BlockSpec((blk, 2 * D), lambda i: (i, 0)),
        out_shape=jax.ShapeDtypeStruct((N, 2 * D), jnp.float32),
    )(x, Wc, bc)


_SC_MESH = plsc.VectorSubcoreMesh(
    core_axis_name="c", subcore_axis_name="s", num_cores=NC, num_subcores=NS
)


_SC_SCRATCH = [
    pltpu.VMEM((R + 1, D), jnp.float32),
    pltpu.VMEM((CH,), jnp.int32),
    pltpu.VMEM((CH + G,), jnp.int32),
    pltpu.VMEM((CH + G,), jnp.int32),
    pltpu.VMEM((G,), jnp.int32),
    pltpu.VMEM((G, D), jnp.float32),
    pltpu.VMEM((CHA, D), jnp.float32),
    pltpu.VMEM((CHA, D), jnp.float32),
    pltpu.SemaphoreType.DMA,
]


def _scatter_max_body(bm_hbm, a_hbm, dst_hbm, out_hbm,
                      s_loc, dst_v, dstl_list, src_list, idx_g, rows_v, a_v,
                      o_v, sem):
    wid = lax.axis_index("s") * NC + lax.axis_index("c")
    lo = wid * R
    lo_v = jnp.full((L,), lo, jnp.int32)
    hi_v = lo_v + R
    neg = jnp.full((L,), -3.0e38, jnp.float32)
    one_v = jnp.full((L,), 1, jnp.int32)
    zero_v = jnp.zeros((L,), jnp.int32)

    @pl.loop(0, R + 1)
    def _init(i):
        for r in range(D // L):
            s_loc[i, pl.ds(r * L, L)] = neg

    @pl.loop(0, NCH)
    def _chunk(c):
        pltpu.sync_copy(dst_hbm.at[pl.ds(c * CH, CH)], dst_v)
        base_src = c * (CH // K)

        def scan_g(g, cnt):
            v = dst_v[pl.ds(g * L, L)]
            m = (v >= lo_v) & (v < hi_v)
            m32 = jnp.where(m, one_v, zero_v)
            pos = cnt + plsc.cumsum(m32) - m32
            plsc.store_scatter(dstl_list, [pos], v - lo_v, mask=m)
            srcv = jnp.full((L,), base_src + g // 2, jnp.int32)
            plsc.store_scatter(src_list, [pos], srcv, mask=m)
            return cnt + plsc.all_reduce_population_count(m)

        cnt_v = lax.fori_loop(0, CH // L, scan_g, jnp.zeros((L,), jnp.int32))
        cnt = cnt_v[0]

        zer = jnp.zeros((L,), jnp.int32)
        dmp = jnp.full((L,), R, jnp.int32)
        for t in range(G // L):
            src_list[pl.ds(cnt + t * L, L)] = zer
            dstl_list[pl.ds(cnt + t * L, L)] = dmp

        nsub = (cnt + (G - 1)) // G

        @pl.loop(0, nsub)
        def _sub(s):
            pltpu.async_copy(
                bm_hbm.at[src_list.at[pl.ds(s * G, G)]], rows_v, sem
            ).wait()
            dv = dstl_list[pl.ds(s * G, L)]
            s_loc[dv[0], pl.ds(0, L)] = rows_v[0, pl.ds(0, L)]

    @pl.loop(0, R // CHA)
    def _fin(t):
        row0 = lo + t * CHA
        pltpu.sync_copy(a_hbm.at[pl.ds(row0, CHA)], a_v)

        @pl.loop(0, CHA)
        def _rowp(e):
            for r in range(D // L):
                v = a_v[e, pl.ds(r * L, L)] + s_loc[t * CHA + e, pl.ds(r * L, L)]
                o_v[e, pl.ds(r * L, L)] = jnp.maximum(v, 0.0)

        pltpu.sync_copy(o_v, out_hbm.at[pl.ds(row0, CHA)])


_scatter_max = pl.kernel(
    _scatter_max_body,
    out_type=jax.ShapeDtypeStruct((NP, D), jnp.float32),
    mesh=_SC_MESH,
    compiler_params=pltpu.CompilerParams(needs_layout_passes=False),
    scratch_types=_SC_SCRATCH,
)


def kernel(x, knn_indices, W, b):
    W1, W2 = W[:D], W[D:]
    Wc = jnp.concatenate([W1 - W2, W2], axis=1)
    bc = jnp.concatenate([b, jnp.zeros_like(b)]).reshape(1, 2 * D)
    AB = _projections(x, Wc, bc)
    A, Bm = AB[:, :D], AB[:, D:]
    A = jnp.pad(A, ((0, NP - N), (0, 0)))
    dst = knn_indices.reshape(-1).astype(jnp.int32)
    dst = jnp.pad(dst, (0, EP - E), constant_values=jnp.int32(1 << 20))
    out = _scatter_max(Bm, A, dst)
    return out[:N]


# speedup vs baseline: 11.7247x; 11.7247x over previous
import jax
import jax.numpy as jnp
from jax import lax
from jax.experimental import pallas as pl
from jax.experimental.pallas import tpu as pltpu
from jax.experimental.pallas import tpu_sc as plsc

N = 10000
K = 32
D = 128

NC = 2
NS = 16
NW = NC * NS
L = 16

R = 320
NP = NW * R
E = N * K
CH = 4096
SRC_CH = CH // K
NCH = -(-E // CH)
EP = NCH * CH
NSP = NCH * SRC_CH
CHA = 64


def _matmul_body(x_ref, wc_ref, bc_ref, out_ref):
    out_ref[...] = (
        jnp.dot(x_ref[...], wc_ref[...], preferred_element_type=jnp.float32)
        + bc_ref[...]
    )


def _projections(x, Wc, bc):
    grid = 10
    blk = N // grid
    return pl.pallas_call(
        _matmul_body,
        grid=(grid,),
        in_specs=[
            pl.BlockSpec((blk, D), lambda i: (i, 0)),
            pl.BlockSpec((D, 2 * D), lambda i: (0, 0)),
            pl.BlockSpec((1, 2 * D), lambda i: (0, 0)),
        ],
        out_specs=pl.BlockSpec((blk, 2 * D), lambda i: (i, 0)),
        out_shape=jax.ShapeDtypeStruct((N, 2 * D), jnp.float32),
    )(x, Wc, bc)


_SC_MESH = plsc.VectorSubcoreMesh(
    core_axis_name="c", subcore_axis_name="s", num_cores=NC, num_subcores=NS
)

_SC_SCRATCH = [
    pltpu.VMEM((R + 1, D), jnp.float32),
    pltpu.VMEM((2 * CH,), jnp.int32),
    pltpu.VMEM((2 * SRC_CH, D), jnp.float32),
    pltpu.VMEM((CH + L,), jnp.int32),
    pltpu.VMEM((CHA, D), jnp.float32),
    pltpu.VMEM((CHA, D), jnp.float32),
    pltpu.SemaphoreType.DMA,
    pltpu.SemaphoreType.DMA,
    pltpu.SemaphoreType.DMA,
    pltpu.SemaphoreType.DMA,
]


def _scatter_max_body(bm_hbm, a_hbm, dst_hbm, out_hbm,
                      s_loc, dst_v2, rows_v2, plist, a_v, o_v,
                      sem_d0, sem_d1, sem_r0, sem_r1):
    wid = lax.axis_index("s") * NC + lax.axis_index("c")
    lo = wid * R
    lo_v = jnp.full((L,), lo, jnp.int32)
    hi_v = lo_v + R
    neg = jnp.full((L,), -3.0e38, jnp.float32)
    one_v = jnp.full((L,), 1, jnp.int32)
    zero_v = jnp.zeros((L,), jnp.int32)
    sem_d = (sem_d0, sem_d1)
    sem_r = (sem_r0, sem_r1)

    @pl.loop(0, R + 1)
    def _init(i):
        for r in range(D // L):
            s_loc[i, pl.ds(r * L, L)] = neg

    def issue(c, buf):
        pltpu.async_copy(dst_hbm.at[pl.ds(c * CH, CH)],
                         dst_v2.at[pl.ds(buf * CH, CH)], sem_d[buf])
        pltpu.async_copy(bm_hbm.at[pl.ds(c * SRC_CH, SRC_CH)],
                         rows_v2.at[pl.ds(buf * SRC_CH, SRC_CH)], sem_r[buf])

    def wait(c, buf):
        pltpu.make_async_copy(dst_hbm.at[pl.ds(c * CH, CH)],
                              dst_v2.at[pl.ds(buf * CH, CH)],
                              sem_d[buf]).wait()
        pltpu.make_async_copy(bm_hbm.at[pl.ds(c * SRC_CH, SRC_CH)],
                              rows_v2.at[pl.ds(buf * SRC_CH, SRC_CH)],
                              sem_r[buf]).wait()

    def process(c, buf):
        def scan_g(g, cnt):
            v = dst_v2[pl.ds(buf * CH + g * L, L)]
            m = (v >= lo_v) & (v < hi_v)
            m32 = jnp.where(m, one_v, zero_v)
            pos = cnt + plsc.cumsum(m32) - m32
            row9 = jnp.full((L,), (g // 2) * 512, jnp.int32)
            plsc.store_scatter(plist, [pos], row9 + (v - lo_v), mask=m)
            return cnt + plsc.all_reduce_population_count(m)

        cnt_v = lax.fori_loop(0, CH // L, scan_g, jnp.zeros((L,), jnp.int32))
        cnt = cnt_v[0]

        iota = lax.broadcasted_iota(jnp.int32, (L,), 0)
        plsc.store_scatter(plist, [cnt_v + iota], jnp.full((L,), R, jnp.int32))

        nt = (cnt + (L - 1)) // L

        @pl.loop(0, nt)
        def _upd(t):
            pv = plist[pl.ds(t * L, L)]
            for j in range(L):
                pj = pv[j]
                dl = lax.rem(pj, 512)
                rl = lax.div(pj, 512)
                for r in range(D // L):
                    sl = s_loc[dl, pl.ds(r * L, L)]
                    rw = rows_v2[buf * SRC_CH + rl, pl.ds(r * L, L)]
                    s_loc[dl, pl.ds(r * L, L)] = jnp.maximum(sl, rw)

    issue(0, 0)

    @pl.loop(0, (NCH + 1) // 2)
    def _pair(p):
        c0 = p * 2

        wait(c0, 0)

        @pl.when(c0 + 1 < NCH)
        def _():
            issue(c0 + 1, 1)

        process(c0, 0)

        @pl.when(c0 + 1 < NCH)
        def _():
            @pl.when(c0 + 2 < NCH)
            def _():
                issue(c0 + 2, 0)

            wait(c0 + 1, 1)
            process(c0 + 1, 1)

    @pl.loop(0, R // CHA)
    def _fin(t):
        row0 = lo + t * CHA
        pltpu.sync_copy(a_hbm.at[pl.ds(row0, CHA)], a_v)

        @pl.loop(0, CHA)
        def _rowp(e):
            for r in range(D // L):
                v = a_v[e, pl.ds(r * L, L)] + s_loc[t * CHA + e, pl.ds(r * L, L)]
                o_v[e, pl.ds(r * L, L)] = jnp.maximum(v, 0.0)

        pltpu.sync_copy(o_v, out_hbm.at[pl.ds(row0, CHA)])


_scatter_max = pl.kernel(
    _scatter_max_body,
    out_type=jax.ShapeDtypeStruct((NP, D), jnp.float32),
    mesh=_SC_MESH,
    compiler_params=pltpu.CompilerParams(needs_layout_passes=False),
    scratch_types=_SC_SCRATCH,
)


def kernel(x, knn_indices, W, b):
    W1, W2 = W[:D], W[D:]
    Wc = jnp.concatenate([W1 - W2, W2], axis=1)
    bc = jnp.concatenate([b, jnp.zeros_like(b)]).reshape(1, 2 * D)
    AB = _projections(x, Wc, bc)
    A, Bm = AB[:, :D], AB[:, D:]
    A = jnp.pad(A, ((0, NP - N), (0, 0)))
    Bm = jnp.pad(Bm, ((0, NSP - N), (0, 0)))
    dst = knn_indices.reshape(-1).astype(jnp.int32)
    dst = jnp.pad(dst, (0, EP - E), constant_values=jnp.int32(1 << 20))
    out = _scatter_max(Bm, A, dst)
    return out[:N]
